# baseline (speedup 1.0000x reference)
import functools

import jax
import jax.numpy as jnp
from jax import lax
from jax.experimental import pallas as pl
from jax.experimental.pallas import tpu as pltpu
from jax.experimental.pallas import tpu_sc as plsc

N = 10000
E = 320000
B = 128
IN = 128
HID = 256
OUT = 128
NC = 4
MAX_HOP = 8

NT = 16
EPT = E // NT
KP = 80
NW = 10
NSB = EPT // (KP * NW)
NP = 10240
NROW = NP // NT

_mesh = plsc.VectorSubcoreMesh(core_axis_name="c", subcore_axis_name="s")


@functools.partial(
    pl.kernel, mesh=_mesh,
    out_type=jax.ShapeDtypeStruct((2 * NP, IN), jnp.float32),
    scratch_types=[
        pltpu.VMEM((NW, KP), jnp.int32),
        pltpu.VMEM((KP, IN), jnp.float32),
        pltpu.VMEM((8, 80), jnp.int32),
        pltpu.VMEM_SHARED((NP, IN), jnp.float32),
        pltpu.SemaphoreType.DMA,
        pltpu.SemaphoreType.DMA,
    ],
)
def _deg_kernel(edge_hbm, out_hbm, idx_v, buf_v, ii_v, deg_sh, isem, ssem):
    c = lax.axis_index("c")
    s = lax.axis_index("s")
    iota = lax.iota(jnp.int32, 16)
    zeros16 = jnp.zeros((16,), jnp.float32)
    ones16 = jnp.ones((16,), jnp.float32)
    r0 = s * NROW
    for r in range(8):
        for k in range(5):
            ii_v[r, pl.ds(k * 16, 16)] = iota + (r0 + 80 * r + 16 * k)
    for i in range(KP):
        for k in range(IN // 16):
            buf_v[i, pl.ds(16 * k, 16)] = zeros16
    for r in range(8):
        pltpu.sync_copy(buf_v, deg_sh.at[ii_v.at[r]])
    plsc.subcore_barrier()

    for i in range(KP):
        for k in range(IN // 16):
            buf_v[i, pl.ds(16 * k, 16)] = ones16

    def _chunk(j, carry):
        base = (1 - c) * E + s * EPT + j * (KP * NW)
        hi = [pltpu.async_copy(edge_hbm.at[pl.ds(base + w * KP, KP)],
                               idx_v.at[w], isem) for w in range(NW)]
        hs = []
        for w in range(NW):
            hi[w].wait()
            hs.append(pltpu.async_copy(buf_v, deg_sh.at[idx_v.at[w]],
                                       ssem, add=True))
        for h in hs:
            h.wait()
        return carry
    lax.fori_loop(0, NSB, _chunk, 0)

    plsc.subcore_barrier()
    for r in range(8):
        pltpu.sync_copy(deg_sh.at[ii_v.at[r]], buf_v)
        pltpu.sync_copy(buf_v, out_hbm.at[pl.ds(c * NP + r0 + 80 * r, 80)])


@functools.partial(
    pl.kernel, mesh=_mesh,
    out_type=jax.ShapeDtypeStruct((2 * NP, IN), jnp.float32),
    scratch_types=[
        pltpu.VMEM((NW * KP,), jnp.int32),
        pltpu.VMEM((NW, KP), jnp.int32),
        pltpu.VMEM((2, KP, IN), jnp.float32),
        pltpu.VMEM((8, 80), jnp.int32),
        pltpu.VMEM_SHARED((NP, IN), jnp.float32),
        pltpu.SemaphoreType.DMA,
        pltpu.SemaphoreType.DMA,
        pltpu.SemaphoreType.DMA,
        pltpu.SemaphoreType.DMA,
        pltpu.SemaphoreType.DMA,
    ],
)
def _prop_kernel(table_hbm, garr_hbm, sarr_hbm, out_hbm, gi_v, si_v, rows_v,
                 ii_v, acc_sh, isem, gsem0, gsem1, ssem0, ssem1):
    c = lax.axis_index("c")
    s = lax.axis_index("s")
    iota = lax.iota(jnp.int32, 16)
    r0 = s * NROW
    for r in range(8):
        for k in range(5):
            ii_v[r, pl.ds(k * 16, 16)] = iota + (r0 + 80 * r + 16 * k)
    for r in range(8):
        pltpu.sync_copy(table_hbm.at[pl.ds(c * NP + r0 + 80 * r, 80)],
                        rows_v.at[0])
        pltpu.sync_copy(rows_v.at[0], acc_sh.at[ii_v.at[r]])
    plsc.subcore_barrier()

    gsem = (gsem0, gsem1)
    ssem = (ssem0, ssem1)

    def _sblock(j, carry):
        base = c * E + s * EPT + j * (KP * NW)
        hgi = pltpu.async_copy(garr_hbm.at[pl.ds(base, KP * NW)], gi_v, isem)
        hsi = [pltpu.async_copy(sarr_hbm.at[pl.ds(base + w * KP, KP)],
                                si_v.at[w], isem) for w in range(NW)]
        hgi.wait()
        for h in hsi:
            h.wait()
        hg = {}
        hs = {}
        hg[0] = pltpu.async_copy(table_hbm.at[gi_v.at[pl.ds(0, KP)]],
                                 rows_v.at[0], gsem[0])
        for w in range(NW):
            b = w & 1
            if w + 1 < NW:
                nb = 1 - b
                if w >= 1:
                    hs[w - 1].wait()
                hg[w + 1] = pltpu.async_copy(
                    table_hbm.at[gi_v.at[pl.ds((w + 1) * KP, KP)]],
                    rows_v.at[nb], gsem[nb])
            hg[w].wait()
            hs[w] = pltpu.async_copy(rows_v.at[b], acc_sh.at[si_v.at[w]],
                                     ssem[b], add=True)
        hs[NW - 2].wait()
        hs[NW - 1].wait()
        return carry
    lax.fori_loop(0, NSB, _sblock, 0)

    plsc.subcore_barrier()
    for r in range(8):
        pltpu.sync_copy(acc_sh.at[ii_v.at[r]], rows_v.at[0])
        pltpu.sync_copy(rows_v.at[0],
                        out_hbm.at[pl.ds(c * NP + r0 + 80 * r, 80)])


def kernel(x, edge_index, batch, user_state, num_hop, params):
    p = params
    u = jnp.sum(user_state, axis=(1, 2))
    alpha = jax.nn.sigmoid(p['raw_alpha'])
    beta = jax.nn.sigmoid(p['raw_beta'])
    u0 = u[:, None]
    U_ = u0 @ p['Wu0'] + p['bu0']
    S_ = jnp.zeros_like(u0) @ p['Ws0'] + p['bs0']
    D_ = jnp.zeros_like(u0) @ p['Wd0'] + p['bd0']
    Ul, Sl, Dl = [], [], []
    for _ in range(MAX_HOP):
        U_ = U_ - alpha * U_ - beta * U_
        U_ = U_ @ p['Wu'] + p['bu']
        S_ = (S_ + alpha * U_) @ p['Ws'] + p['bs']
        D_ = (D_ + beta * U_) @ p['Wd'] + p['bd']
        Ul.append(U_); Sl.append(S_); Dl.append(D_)
    U = jnp.stack(Ul, axis=1)
    S = jnp.stack(Sl, axis=1)
    D = jnp.stack(Dl, axis=1)
    hop = jnp.clip(num_hop.astype(jnp.int32) - 1, 0, MAX_HOP - 1)
    bidx = jnp.arange(B)
    xg = jnp.concatenate([U[bidx, hop], S[bidx, hop], D[bidx, hop]], axis=1) @ p['Wx'] + p['bx']
    Uo = U @ p['lu'] + p['blu']
    So = S @ p['ls'] + p['bls']
    Do = D @ p['ld'] + p['bld']

    onehot = (batch[:, None] == jnp.arange(B, dtype=batch.dtype)[None, :]).astype(jnp.float32)
    cnt = jnp.sum(onehot, axis=0)
    first = jnp.clip(jnp.cumsum(cnt).astype(jnp.int32) - cnt.astype(jnp.int32), 0, N - 1)
    nonempty = (cnt > 0).astype(jnp.float32)
    x_first_relu = jax.nn.relu(x[first])

    edge_flat = edge_index.reshape(-1)
    degs = _deg_kernel(edge_flat)
    deg_td = degs[:N, 0] + 1.0
    deg_bu = degs[NP:NP + N, 0] + 1.0
    dis_td = lax.rsqrt(deg_td)[:, None]
    dis_bu = lax.rsqrt(deg_bu)[:, None]

    src, dst = edge_index[0], edge_index[1]
    garr = jnp.concatenate([src, dst + NP])
    sarr = jnp.concatenate([dst, src])
    padrows = jnp.zeros((NP - N, IN), jnp.float32)
    tab1 = jnp.concatenate([x * dis_td, padrows, x * dis_bu, padrows], axis=0)
    A1 = _prop_kernel(tab1, garr, sarr)
    out1_td = (A1[:N] * dis_td) @ p['td_W1'] + p['td_b1']
    out1_bu = (A1[NP:NP + N] * dis_bu) @ p['bu_W1'] + p['bu_b1']

    q_td = jax.nn.relu(out1_td) @ p['td_W2'][:HID] + onehot @ (x_first_relu @ p['td_W2'][HID:])
    q_bu = jax.nn.relu(out1_bu) @ p['bu_W2'][:HID] + onehot @ (x_first_relu @ p['bu_W2'][HID:])

    tab2 = jnp.concatenate([q_td * dis_td, padrows, q_bu * dis_bu, padrows], axis=0)
    A2 = _prop_kernel(tab2, garr, sarr)
    out2_td = jax.nn.relu(A2[:N] * dis_td + p['td_b2'])
    out2_bu = jax.nn.relu(A2[NP:NP + N] * dis_bu + p['bu_b2'])

    inv_cnt = 1.0 / jnp.maximum(cnt, 1.0)[:, None]
    TD = jnp.concatenate([(onehot.T @ out2_td) * inv_cnt,
                          out1_td[first] * nonempty[:, None]], axis=1) + xg
    BU = jnp.concatenate([(onehot.T @ out2_bu) * inv_cnt,
                          out1_bu[first] * nonempty[:, None]], axis=1) + xg

    logits = jnp.concatenate([BU, TD], axis=1) @ p['fcW'] + p['fcb']
    return (jax.nn.log_softmax(logits, axis=-1), Uo, So, Do)
